# baseline (speedup 1.0000x reference)
import functools

import jax
import jax.numpy as jnp
from jax import lax
from jax.experimental import pallas as pl
from jax.experimental.pallas import tpu as pltpu
from jax.experimental.pallas import tpu_sc as plsc

_SEQ = 200
_BATCH = 4096
_EMBED = 64
_HIDDEN = 256
_OUT = 2
_VOCAB = 1000000

_NC = 2
_NS = 16
_NW = _NC * _NS
_BPW = _BATCH // _NW
_NBUF = 10
_STEPS = _SEQ // _NBUF

_UROWS = 256
_NUNITS = _VOCAB // _UROWS
_TAIL_ROWS = _VOCAB - _NUNITS * _UROWS
_UPW = _NUNITS // _NW
_EXTRA = _NUNITS - _UPW * _NW

_mesh = plsc.VectorSubcoreMesh(core_axis_name="c", subcore_axis_name="s")


@functools.partial(
    pl.kernel,
    mesh=_mesh,
    compiler_params=pltpu.CompilerParams(needs_layout_passes=False),
    out_type=jax.ShapeDtypeStruct((_VOCAB * _EMBED,), jnp.float32),
    scratch_types=[
        pltpu.VMEM((_EMBED, _UROWS), jnp.float32),
        pltpu.VMEM((_EMBED, _UROWS), jnp.float32),
        pltpu.VMEM((_UROWS * _EMBED,), jnp.float32),
        pltpu.VMEM((_UROWS * _EMBED,), jnp.float32),
        pltpu.VMEM((_TAIL_ROWS, _EMBED), jnp.float32),
        pltpu.SemaphoreType.DMA,
        pltpu.SemaphoreType.DMA,
        pltpu.SemaphoreType.DMA,
        pltpu.SemaphoreType.DMA,
    ],
)
def _detile(tT_hbm, tail_hbm, flat_hbm, in0, in1, out0, out1, tail_v,
            si0, si1, so0, so1):
    wid = lax.axis_index("s") * _NC + lax.axis_index("c")
    nu = _UPW + jnp.where(wid < _EXTRA, 1, 0)
    inb = (in0, in1)
    outb = (out0, out1)
    sin = (si0, si1)
    sout = (so0, so1)
    lanes = lax.iota(jnp.int32, 16)

    def unit_of(i):
        return wid + i * _NW

    def in_copy(i, p):
        return pltpu.make_async_copy(
            tT_hbm.at[:, pl.ds(unit_of(i) * _UROWS, _UROWS)], inb[p], sin[p]
        )

    def out_copy(i, p):
        return pltpu.make_async_copy(
            outb[p],
            flat_hbm.at[pl.ds(unit_of(i) * _UROWS * _EMBED, _UROWS * _EMBED)],
            sout[p],
        )

    def transpose_rows(p, nrows):
        def row(r, c):
            col = jnp.full((16,), r, jnp.int32)
            for j in range(_EMBED // 16):
                v = plsc.load_gather(inb[p], [lanes + j * 16, col])
                outb[p][pl.ds(r * _EMBED + j * 16, 16)] = v
            return c

        lax.fori_loop(0, nrows, row, 0, unroll=8)

    in_copy(0, 0).start()
    in_copy(1, 1).start()

    def do_unit(i, p):
        in_copy(i, p).wait()

        @pl.when(i >= 2)
        def _():
            out_copy(i - 2, p).wait()

        transpose_rows(p, _UROWS)
        out_copy(i, p).start()

        @pl.when(i + 2 < nu)
        def _():
            in_copy(i + 2, p).start()

    def body(g, carry):
        for p in range(2):
            i = 2 * g + p

            @pl.when(i < nu)
            def _():
                do_unit(i, p)
        return carry

    lax.fori_loop(0, (_UPW + 2) // 2, body, 0, unroll=False)

    out_copy(0, 0).wait()
    out_copy(0, 1).wait()

    @pl.when(wid == _NW - 1)
    def _():
        base = _NUNITS * _UROWS
        pltpu.sync_copy(tail_hbm, tail_v)

        def row(r, c):
            for j in range(_EMBED // 16):
                out0[pl.ds(r * _EMBED + j * 16, 16)] = tail_v[
                    r, pl.ds(j * 16, 16)
                ]
            return c

        lax.fori_loop(0, _TAIL_ROWS, row, 0, unroll=8)
        pltpu.sync_copy(
            out0.at[pl.ds(0, _TAIL_ROWS * _EMBED)],
            flat_hbm.at[pl.ds(base * _EMBED, _TAIL_ROWS * _EMBED)],
        )


@functools.partial(
    pl.kernel,
    mesh=_mesh,
    compiler_params=pltpu.CompilerParams(use_tc_tiling_on_sc=False),
    out_type=jax.ShapeDtypeStruct((_BATCH, _EMBED), jnp.float32),
    scratch_types=[
        pltpu.VMEM((_SEQ, _BPW), jnp.int32),
        pltpu.VMEM((_NBUF, _BPW, _EMBED), jnp.float32),
        pltpu.VMEM((_BPW, _EMBED), jnp.float32),
    ]
    + [pltpu.SemaphoreType.DMA] * _NBUF,
)
def _sc_pool(x_hbm, table_hbm, out_hbm, idx_v, acc_v, sum_v, *sems):
    wid = lax.axis_index("s") * _NC + lax.axis_index("c")
    base = wid * _BPW

    pltpu.sync_copy(x_hbm.at[:, pl.ds(base, _BPW)], idx_v)

    for b in range(_NBUF):
        pltpu.async_copy(table_hbm.at[idx_v.at[b]], acc_v.at[b], sems[b])

    def step(g, carry):
        for b in range(_NBUF):
            s = g * _NBUF + b
            pltpu.make_async_copy(
                table_hbm.at[idx_v.at[s]], acc_v.at[b], sems[b]
            ).wait()
            pltpu.async_copy(
                table_hbm.at[idx_v.at[s]], acc_v.at[b], sems[b], add=True
            )
        return carry

    lax.fori_loop(1, _STEPS, step, 0, unroll=False)

    for b in range(_NBUF):
        pltpu.make_async_copy(
            table_hbm.at[idx_v.at[b]], acc_v.at[b], sems[b]
        ).wait()

    def combine(r, carry):
        for j in range(_EMBED // 16):
            sl = pl.ds(j * 16, 16)
            v = acc_v[0, r, sl]
            for b in range(1, _NBUF):
                v = v + acc_v[b, r, sl]
            sum_v[r, sl] = v
        return carry

    lax.fori_loop(0, _BPW, combine, 0, unroll=False)

    pltpu.sync_copy(sum_v, out_hbm.at[pl.ds(base, _BPW)])


def _mlp_body(sums_ref, w1_ref, b1_ref, w2_ref, b2_ref, out_ref):
    h = jnp.dot(sums_ref[...], w1_ref[...], preferred_element_type=jnp.float32)
    h = jnp.maximum(h + b1_ref[...], 0.0)
    out_ref[...] = (
        jnp.dot(h, w2_ref[...], preferred_element_type=jnp.float32)
        + b2_ref[...]
    )


_BB = 512


def _mlp(sums, w1s, b1, w2, b2):
    return pl.pallas_call(
        _mlp_body,
        out_shape=jax.ShapeDtypeStruct((_BATCH, _OUT), jnp.float32),
        grid=(_BATCH // _BB,),
        in_specs=[
            pl.BlockSpec((_BB, _EMBED), lambda i: (i, 0)),
            pl.BlockSpec((_EMBED, _HIDDEN), lambda i: (0, 0)),
            pl.BlockSpec((1, _HIDDEN), lambda i: (0, 0)),
            pl.BlockSpec((_HIDDEN, _OUT), lambda i: (0, 0)),
            pl.BlockSpec((1, _OUT), lambda i: (0, 0)),
        ],
        out_specs=pl.BlockSpec((_BB, _OUT), lambda i: (i, 0)),
    )(sums, w1s, b1, w2, b2)


def kernel(x, table, W1, b1, W2, b2):
    tail = lax.slice(table, (_NUNITS * _UROWS, 0), (_VOCAB, _EMBED))
    flat = _detile(table.T, tail)
    table_lin = flat.reshape(_VOCAB, _EMBED)
    sums = _sc_pool(x, table_lin)
    w1s = W1 * jnp.float32(1.0 / _SEQ)
    return _mlp(sums, w1s, b1.reshape(1, _HIDDEN), W2, b2.reshape(1, _OUT))


# speedup vs baseline: 1.2150x; 1.2150x over previous
import functools

import jax
import jax.numpy as jnp
from jax import lax
from jax.experimental import pallas as pl
from jax.experimental.pallas import tpu as pltpu
from jax.experimental.pallas import tpu_sc as plsc

_SEQ = 200
_BATCH = 4096
_EMBED = 64
_HIDDEN = 256
_OUT = 2
_VOCAB = 1000000

_NC = 2
_NS = 16
_NW = _NC * _NS
_BPW = _BATCH // _NW
_NBUF = 10
_STEPS = _SEQ // _NBUF

_UROWS = 256
_NUNITS = _VOCAB // _UROWS
_TAIL_ROWS = _VOCAB - _NUNITS * _UROWS
_UPW = _NUNITS // _NW
_EXTRA = _NUNITS - _UPW * _NW

_mesh = plsc.VectorSubcoreMesh(core_axis_name="c", subcore_axis_name="s")


@functools.partial(
    pl.kernel,
    mesh=_mesh,
    compiler_params=pltpu.CompilerParams(needs_layout_passes=False),
    out_type=jax.ShapeDtypeStruct((_VOCAB * _EMBED,), jnp.float32),
    scratch_types=[
        pltpu.VMEM((_EMBED, _UROWS), jnp.float32),
        pltpu.VMEM((_EMBED, _UROWS), jnp.float32),
        pltpu.VMEM((_UROWS * _EMBED,), jnp.float32),
        pltpu.VMEM((_UROWS * _EMBED,), jnp.float32),
        pltpu.VMEM((_TAIL_ROWS, _EMBED), jnp.float32),
        pltpu.SemaphoreType.DMA,
        pltpu.SemaphoreType.DMA,
        pltpu.SemaphoreType.DMA,
        pltpu.SemaphoreType.DMA,
    ],
)
def _detile(tT_hbm, tail_hbm, flat_hbm, in0, in1, out0, out1, tail_v,
            si0, si1, so0, so1):
    wid = lax.axis_index("s") * _NC + lax.axis_index("c")
    nu = _UPW + jnp.where(wid < _EXTRA, 1, 0)
    inb = (in0, in1)
    outb = (out0, out1)
    sin = (si0, si1)
    sout = (so0, so1)
    lanes = lax.iota(jnp.int32, 16)

    def unit_of(i):
        return wid + i * _NW

    def in_copy(i, p):
        return pltpu.make_async_copy(
            tT_hbm.at[:, pl.ds(unit_of(i) * _UROWS, _UROWS)], inb[p], sin[p]
        )

    def out_copy(i, p):
        return pltpu.make_async_copy(
            outb[p],
            flat_hbm.at[pl.ds(unit_of(i) * _UROWS * _EMBED, _UROWS * _EMBED)],
            sout[p],
        )

    def transpose_rows(p, nrows):
        scat = lanes * _EMBED

        def col(c, carry):
            for g in range(nrows // 16):
                v = inb[p][c, pl.ds(g * 16, 16)]
                plsc.store_scatter(
                    outb[p], [scat + (g * 16 * _EMBED + c)], v
                )
            return carry

        lax.fori_loop(0, _EMBED, col, 0, unroll=4)

    in_copy(0, 0).start()
    in_copy(1, 1).start()

    def do_unit(i, p):
        in_copy(i, p).wait()

        @pl.when(i >= 2)
        def _():
            out_copy(i - 2, p).wait()

        transpose_rows(p, _UROWS)
        out_copy(i, p).start()

        @pl.when(i + 2 < nu)
        def _():
            in_copy(i + 2, p).start()

    def body(g, carry):
        for p in range(2):
            i = 2 * g + p

            @pl.when(i < nu)
            def _():
                do_unit(i, p)
        return carry

    lax.fori_loop(0, (_UPW + 2) // 2, body, 0, unroll=False)

    out_copy(0, 0).wait()
    out_copy(0, 1).wait()

    @pl.when(wid == _NW - 1)
    def _():
        base = _NUNITS * _UROWS
        pltpu.sync_copy(tail_hbm, tail_v)

        def row(r, c):
            for j in range(_EMBED // 16):
                out0[pl.ds(r * _EMBED + j * 16, 16)] = tail_v[
                    r, pl.ds(j * 16, 16)
                ]
            return c

        lax.fori_loop(0, _TAIL_ROWS, row, 0, unroll=8)
        pltpu.sync_copy(
            out0.at[pl.ds(0, _TAIL_ROWS * _EMBED)],
            flat_hbm.at[pl.ds(base * _EMBED, _TAIL_ROWS * _EMBED)],
        )


@functools.partial(
    pl.kernel,
    mesh=_mesh,
    compiler_params=pltpu.CompilerParams(use_tc_tiling_on_sc=False),
    out_type=jax.ShapeDtypeStruct((_BATCH, _EMBED), jnp.float32),
    scratch_types=[
        pltpu.VMEM((_SEQ, _BPW), jnp.int32),
        pltpu.VMEM((_NBUF, _BPW, _EMBED), jnp.float32),
        pltpu.VMEM((_BPW, _EMBED), jnp.float32),
    ]
    + [pltpu.SemaphoreType.DMA] * _NBUF,
)
def _sc_pool(x_hbm, table_hbm, out_hbm, idx_v, acc_v, sum_v, *sems):
    wid = lax.axis_index("s") * _NC + lax.axis_index("c")
    base = wid * _BPW

    pltpu.sync_copy(x_hbm.at[:, pl.ds(base, _BPW)], idx_v)

    for b in range(_NBUF):
        pltpu.async_copy(table_hbm.at[idx_v.at[b]], acc_v.at[b], sems[b])

    def step(g, carry):
        for b in range(_NBUF):
            s = g * _NBUF + b
            pltpu.make_async_copy(
                table_hbm.at[idx_v.at[s]], acc_v.at[b], sems[b]
            ).wait()
            pltpu.async_copy(
                table_hbm.at[idx_v.at[s]], acc_v.at[b], sems[b], add=True
            )
        return carry

    lax.fori_loop(1, _STEPS, step, 0, unroll=False)

    for b in range(_NBUF):
        pltpu.make_async_copy(
            table_hbm.at[idx_v.at[b]], acc_v.at[b], sems[b]
        ).wait()

    def combine(r, carry):
        for j in range(_EMBED // 16):
            sl = pl.ds(j * 16, 16)
            v = acc_v[0, r, sl]
            for b in range(1, _NBUF):
                v = v + acc_v[b, r, sl]
            sum_v[r, sl] = v
        return carry

    lax.fori_loop(0, _BPW, combine, 0, unroll=False)

    pltpu.sync_copy(sum_v, out_hbm.at[pl.ds(base, _BPW)])


def _mlp_body(sums_ref, w1_ref, b1_ref, w2_ref, b2_ref, out_ref):
    h = jnp.dot(sums_ref[...], w1_ref[...], preferred_element_type=jnp.float32)
    h = jnp.maximum(h + b1_ref[...], 0.0)
    out_ref[...] = (
        jnp.dot(h, w2_ref[...], preferred_element_type=jnp.float32)
        + b2_ref[...]
    )


_BB = 512


def _mlp(sums, w1s, b1, w2, b2):
    return pl.pallas_call(
        _mlp_body,
        out_shape=jax.ShapeDtypeStruct((_BATCH, _OUT), jnp.float32),
        grid=(_BATCH // _BB,),
        in_specs=[
            pl.BlockSpec((_BB, _EMBED), lambda i: (i, 0)),
            pl.BlockSpec((_EMBED, _HIDDEN), lambda i: (0, 0)),
            pl.BlockSpec((1, _HIDDEN), lambda i: (0, 0)),
            pl.BlockSpec((_HIDDEN, _OUT), lambda i: (0, 0)),
            pl.BlockSpec((1, _OUT), lambda i: (0, 0)),
        ],
        out_specs=pl.BlockSpec((_BB, _OUT), lambda i: (i, 0)),
    )(sums, w1s, b1, w2, b2)


def kernel(x, table, W1, b1, W2, b2):
    tail = lax.slice(table, (_NUNITS * _UROWS, 0), (_VOCAB, _EMBED))
    flat = _detile(table.T, tail)
    table_lin = flat.reshape(_VOCAB, _EMBED)
    sums = _sc_pool(x, table_lin)
    w1s = W1 * jnp.float32(1.0 / _SEQ)
    return _mlp(sums, w1s, b1.reshape(1, _HIDDEN), W2, b2.reshape(1, _OUT))


# speedup vs baseline: 2.2923x; 1.8867x over previous
import functools

import jax
import jax.numpy as jnp
from jax import lax
from jax.experimental import pallas as pl
from jax.experimental.pallas import tpu as pltpu
from jax.experimental.pallas import tpu_sc as plsc

_SEQ = 200
_BATCH = 4096
_EMBED = 64
_HIDDEN = 256
_OUT = 2

_NC = 2
_NS = 16
_NW = _NC * _NS
_BPW = _BATCH // _NW
_NBUF = 10
_STEPS = _SEQ // _NBUF

_mesh = plsc.VectorSubcoreMesh(core_axis_name="c", subcore_axis_name="s")


@functools.partial(
    pl.kernel,
    mesh=_mesh,
    compiler_params=pltpu.CompilerParams(use_tc_tiling_on_sc=False),
    out_type=jax.ShapeDtypeStruct((_BATCH, _EMBED), jnp.float32),
    scratch_types=[
        pltpu.VMEM((_SEQ, _BPW), jnp.int32),
        pltpu.VMEM((_NBUF, _BPW, _EMBED), jnp.float32),
        pltpu.VMEM((_BPW, _EMBED), jnp.float32),
    ]
    + [pltpu.SemaphoreType.DMA] * _NBUF,
)
def _sc_pool(x_hbm, table_hbm, out_hbm, idx_v, acc_v, sum_v, *sems):
    wid = lax.axis_index("s") * _NC + lax.axis_index("c")
    base = wid * _BPW

    pltpu.sync_copy(x_hbm.at[:, pl.ds(base, _BPW)], idx_v)

    for b in range(_NBUF):
        pltpu.async_copy(table_hbm.at[idx_v.at[b]], acc_v.at[b], sems[b])

    def step(g, carry):
        for b in range(_NBUF):
            s = g * _NBUF + b
            pltpu.make_async_copy(
                table_hbm.at[idx_v.at[s]], acc_v.at[b], sems[b]
            ).wait()
            pltpu.async_copy(
                table_hbm.at[idx_v.at[s]], acc_v.at[b], sems[b], add=True
            )
        return carry

    lax.fori_loop(1, _STEPS, step, 0, unroll=False)

    for b in range(_NBUF):
        pltpu.make_async_copy(
            table_hbm.at[idx_v.at[b]], acc_v.at[b], sems[b]
        ).wait()

    def combine(r, carry):
        for j in range(_EMBED // 16):
            sl = pl.ds(j * 16, 16)
            v = acc_v[0, r, sl]
            for b in range(1, _NBUF):
                v = v + acc_v[b, r, sl]
            sum_v[r, sl] = v
        return carry

    lax.fori_loop(0, _BPW, combine, 0, unroll=False)

    pltpu.sync_copy(sum_v, out_hbm.at[pl.ds(base, _BPW)])


def _mlp_body(sums_ref, w1_ref, b1_ref, w2_ref, b2_ref, out_ref):
    h = jnp.dot(sums_ref[...], w1_ref[...], preferred_element_type=jnp.float32)
    h = jnp.maximum(h + b1_ref[...], 0.0)
    out_ref[...] = (
        jnp.dot(h, w2_ref[...], preferred_element_type=jnp.float32)
        + b2_ref[...]
    )


_BB = 512


def _mlp(sums, w1s, b1, w2, b2):
    return pl.pallas_call(
        _mlp_body,
        out_shape=jax.ShapeDtypeStruct((_BATCH, _OUT), jnp.float32),
        grid=(_BATCH // _BB,),
        in_specs=[
            pl.BlockSpec((_BB, _EMBED), lambda i: (i, 0)),
            pl.BlockSpec((_EMBED, _HIDDEN), lambda i: (0, 0)),
            pl.BlockSpec((1, _HIDDEN), lambda i: (0, 0)),
            pl.BlockSpec((_HIDDEN, _OUT), lambda i: (0, 0)),
            pl.BlockSpec((1, _OUT), lambda i: (0, 0)),
        ],
        out_specs=pl.BlockSpec((_BB, _OUT), lambda i: (i, 0)),
    )(sums, w1s, b1, w2, b2)


def kernel(x, table, W1, b1, W2, b2):
    sums = _sc_pool(x, table)
    w1s = W1 * jnp.float32(1.0 / _SEQ)
    return _mlp(sums, w1s, b1.reshape(1, _HIDDEN), W2, b2.reshape(1, _OUT))
